# baseline (speedup 1.0000x reference)
import functools

import jax
import jax.numpy as jnp
from jax import lax
from jax.experimental import pallas as pl
from jax.experimental.pallas import tpu as pltpu
from jax.experimental.pallas import tpu_sc as plsc

_B, _N, _M = 16, 2048, 64
_SC_L = 16


def _tc_body(dist_ref, pt_ref, coef_ref, out_ref, epa_ref):
    d = dist_ref[0]
    pt = pt_ref[0]
    is1 = pt == 1
    is2 = pt == 2

    def sel(i):
        return jnp.where(is1, coef_ref[i, 1],
                         jnp.where(is2, coef_ref[i, 2], coef_ref[i, 0]))

    x = jnp.clip(sel(5) * d - sel(4), 0.0, 1.0)
    x3 = x * x * x
    fc = 1.0 - x3 * (x * (6.0 * x - 15.0) + 10.0)

    half_phi = jnp.exp2(sel(0) - sel(1) * d) * fc
    rho_e = jnp.exp2(sel(2) - sel(3) * d) * fc

    half_phi_sum = jnp.sum(half_phi)
    s = jnp.sum(rho_e, axis=0, keepdims=True) + 1e-12
    emb_sum = jnp.sum(s * jax.lax.rsqrt(s))

    e = half_phi_sum - emb_sum
    out_ref[0] = jnp.full((1, 128), e, jnp.float32)
    epa_ref[0] = jnp.full((1, 128), e * (1.0 / _N), jnp.float32)


def _sc_type_counts(types):
    info = plsc.get_sparse_core_info()
    nc = info.num_cores

    mesh = plsc.VectorSubcoreMesh(core_axis_name="c", subcore_axis_name="s")

    @functools.partial(
        pl.kernel, mesh=mesh,
        out_type=jax.ShapeDtypeStruct((_B, _SC_L), jnp.int32),
        scratch_types=[
            pltpu.VMEM((_N,), jnp.int32),
            pltpu.VMEM((_SC_L,), jnp.int32),
        ],
    )
    def counts_kernel(types_hbm, out_hbm, row_v, acc_v):
        wid = lax.axis_index("s") * nc + lax.axis_index("c")

        @pl.when(wid < _B)
        def _():
            pltpu.sync_copy(types_hbm.at[wid], row_v)
            acc = jnp.zeros((_SC_L,), jnp.int32)
            for i in range(_N // _SC_L):
                acc += row_v[pl.ds(i * _SC_L, _SC_L)]
            acc_v[...] = acc
            pltpu.sync_copy(acc_v, out_hbm.at[wid])

    return counts_kernel(types)


def kernel(types, pair_types, distances, A, xi, p, q, r0, offset, cut_a, cut_b):
    dist_t = distances.transpose(0, 2, 1)
    pt_t = pair_types.transpose(0, 2, 1)

    inv_ln2 = 1.4426950408889634
    inv_ba = 1.0 / (cut_b - cut_a)
    coef = jnp.concatenate([
        jnp.stack([
            jnp.log2(0.5 * A) + p * inv_ln2,
            (p / r0) * inv_ln2,
            2.0 * jnp.log2(xi) + 2.0 * q * inv_ln2,
            (2.0 * q / r0) * inv_ln2,
            cut_a * inv_ba,
            inv_ba,
        ]),
        jnp.pad(offset, (0, 1)).reshape(1, 3),
    ])

    count1 = _sc_type_counts(types)

    energy_raw, epa_raw = pl.pallas_call(
        _tc_body,
        compiler_params=pltpu.CompilerParams(
            dimension_semantics=("parallel",)),
        grid=(_B,),
        in_specs=[
            pl.BlockSpec((1, _M, _N), lambda b: (b, 0, 0)),
            pl.BlockSpec((1, _M, _N), lambda b: (b, 0, 0)),
            pl.BlockSpec((7, 3), lambda b: (0, 0)),
        ],
        out_specs=[
            pl.BlockSpec((1, 1, 128), lambda b: (b, 0, 0)),
            pl.BlockSpec((1, 1, 128), lambda b: (b, 0, 0)),
        ],
        out_shape=[
            jax.ShapeDtypeStruct((_B, 1, 128), jnp.float32),
            jax.ShapeDtypeStruct((_B, 1, 128), jnp.float32),
        ],
    )(dist_t, pt_t, coef)

    n1 = jnp.sum(count1, axis=1, keepdims=True).astype(jnp.float32)
    off_sum = offset[0] * (_N - n1) + offset[1] * n1
    energy = energy_raw[:, 0, :1] + off_sum
    energy_per_atom = epa_raw[:, 0, :1] + off_sum * (1.0 / _N)
    return (energy, energy_per_atom)


# speedup vs baseline: 1.5476x; 1.5476x over previous
import jax
import jax.numpy as jnp
from jax.experimental import pallas as pl
from jax.experimental.pallas import tpu as pltpu

_B, _N, _M = 16, 2048, 64


def _tc_body(dist_ref, pt_ref, coef_ref, out_ref, epa_ref):
    d = dist_ref[0]
    pt = pt_ref[0]
    is1 = pt == 1
    is2 = pt == 2

    def sel(i):
        return jnp.where(is1, coef_ref[i, 1],
                         jnp.where(is2, coef_ref[i, 2], coef_ref[i, 0]))

    x = jnp.clip(sel(5) * d - sel(4), 0.0, 1.0)
    x3 = x * x * x
    fc = 1.0 - x3 * (x * (6.0 * x - 15.0) + 10.0)

    half_phi = jnp.exp2(sel(0) - sel(1) * d) * fc
    rho_e = jnp.exp2(sel(2) - sel(3) * d) * fc

    half_phi_sum = jnp.sum(half_phi)
    s = jnp.sum(rho_e, axis=0, keepdims=True) + 1e-12
    emb_sum = jnp.sum(s * jax.lax.rsqrt(s))

    e = half_phi_sum - emb_sum
    out_ref[0] = jnp.full((1, 128), e, jnp.float32)
    epa_ref[0] = jnp.full((1, 128), e * (1.0 / _N), jnp.float32)


def kernel(types, pair_types, distances, A, xi, p, q, r0, offset, cut_a, cut_b):
    dist_t = distances.transpose(0, 2, 1)
    pt_t = pair_types.transpose(0, 2, 1)

    inv_ln2 = 1.4426950408889634
    inv_ba = 1.0 / (cut_b - cut_a)
    coef = jnp.concatenate([
        jnp.stack([
            jnp.log2(0.5 * A) + p * inv_ln2,
            (p / r0) * inv_ln2,
            2.0 * jnp.log2(xi) + 2.0 * q * inv_ln2,
            (2.0 * q / r0) * inv_ln2,
            cut_a * inv_ba,
            inv_ba,
        ]),
        jnp.pad(offset, (0, 1)).reshape(1, 3),
    ])

    energy_raw, epa_raw = pl.pallas_call(
        _tc_body,
        compiler_params=pltpu.CompilerParams(
            dimension_semantics=("parallel",)),
        grid=(_B,),
        in_specs=[
            pl.BlockSpec((1, _M, _N), lambda b: (b, 0, 0)),
            pl.BlockSpec((1, _M, _N), lambda b: (b, 0, 0)),
            pl.BlockSpec((7, 3), lambda b: (0, 0)),
        ],
        out_specs=[
            pl.BlockSpec((1, 1, 128), lambda b: (b, 0, 0)),
            pl.BlockSpec((1, 1, 128), lambda b: (b, 0, 0)),
        ],
        out_shape=[
            jax.ShapeDtypeStruct((_B, 1, 128), jnp.float32),
            jax.ShapeDtypeStruct((_B, 1, 128), jnp.float32),
        ],
    )(dist_t, pt_t, coef)

    off_sum = jnp.sum(jnp.where(types == 1, offset[1], offset[0]),
                      axis=1, keepdims=True)
    energy = energy_raw[:, 0, :1] + off_sum
    energy_per_atom = epa_raw[:, 0, :1] + off_sum * (1.0 / _N)
    return (energy, energy_per_atom)


# speedup vs baseline: 1.8027x; 1.1649x over previous
import jax
import jax.numpy as jnp
from jax.experimental import pallas as pl
from jax.experimental.pallas import tpu as pltpu

_B, _N, _M = 16, 2048, 64
_ILN2 = 1.4426950408889634


def _body(dist_ref, pt_ref, types_ref, a_ref, xi_ref, p_ref, q_ref, r0_ref,
          off_ref, ca_ref, cb_ref, out_ref, epa_ref, coef_s):
    b = pl.program_id(0)

    av = a_ref[...]
    pv = p_ref[...]
    qv = q_ref[...]
    r0v = r0_ref[...]
    xiv = xi_ref[...]
    cav = ca_ref[...]
    inv_ba = 1.0 / (cb_ref[...] - cav)
    coef_s[0:1, :] = jnp.log2(0.5 * av) + pv * _ILN2
    coef_s[1:2, :] = (pv / r0v) * _ILN2
    coef_s[2:3, :] = 2.0 * jnp.log2(xiv) + 2.0 * qv * _ILN2
    coef_s[3:4, :] = (2.0 * qv / r0v) * _ILN2
    coef_s[4:5, :] = cav * inv_ba
    coef_s[5:6, :] = inv_ba

    d = dist_ref[0]
    pt = pt_ref[0]
    is1 = pt == 1
    is2 = pt == 2

    def sel(i):
        return jnp.where(is1, coef_s[i, 1],
                         jnp.where(is2, coef_s[i, 2], coef_s[i, 0]))

    x = jnp.clip(sel(5) * d - sel(4), 0.0, 1.0)
    x3 = x * x * x
    fc = 1.0 - x3 * (x * (6.0 * x - 15.0) + 10.0)

    half_phi = jnp.exp2(sel(0) - sel(1) * d) * fc
    rho_e = jnp.exp2(sel(2) - sel(3) * d) * fc

    half_phi_sum = jnp.sum(half_phi)
    s = jnp.sum(rho_e, axis=0, keepdims=True) + 1e-12

    trow = types_ref[0]
    off_vec = jnp.where(trow == 1, off_ref[0, 1], off_ref[0, 0])
    emb_off_sum = jnp.sum(s * jax.lax.rsqrt(s) - off_vec)

    e = half_phi_sum - emb_off_sum
    bmask = jax.lax.broadcasted_iota(jnp.int32, (1, _B), 1) == b
    erow = jnp.where(bmask, e, out_ref[...])
    out_ref[...] = erow

    @pl.when(b == _B - 1)
    def _fin():
        epa_ref[...] = erow * (1.0 / _N)


def kernel(types, pair_types, distances, A, xi, p, q, r0, offset, cut_a, cut_b):
    dist_t = distances.transpose(0, 2, 1)
    pt_t = pair_types.transpose(0, 2, 1)
    types3 = types.reshape(_B, 1, _N)

    full = lambda shape: pl.BlockSpec(shape, lambda b: tuple(0 for _ in shape))
    energy, energy_per_atom = pl.pallas_call(
        _body,
        grid=(_B,),
        in_specs=[
            pl.BlockSpec((1, _M, _N), lambda b: (b, 0, 0)),
            pl.BlockSpec((1, _M, _N), lambda b: (b, 0, 0)),
            pl.BlockSpec((1, 1, _N), lambda b: (b, 0, 0)),
            full((1, 3)), full((1, 3)), full((1, 3)), full((1, 3)),
            full((1, 3)), full((1, 2)), full((1, 3)), full((1, 3)),
        ],
        out_specs=[
            full((1, _B)),
            full((1, _B)),
        ],
        out_shape=[
            jax.ShapeDtypeStruct((1, _B), jnp.float32),
            jax.ShapeDtypeStruct((1, _B), jnp.float32),
        ],
        scratch_shapes=[pltpu.VMEM((7, 3), jnp.float32)],
    )(dist_t, pt_t, types3,
      A.reshape(1, 3), xi.reshape(1, 3), p.reshape(1, 3), q.reshape(1, 3),
      r0.reshape(1, 3), offset.reshape(1, 2), cut_a.reshape(1, 3),
      cut_b.reshape(1, 3))

    return (energy.T, energy_per_atom.T)


# speedup vs baseline: 2.1471x; 1.1910x over previous
import jax
import jax.numpy as jnp
from jax.experimental import pallas as pl
from jax.experimental.pallas import tpu as pltpu

_B, _N, _M = 16, 2048, 64
_ILN2 = 1.4426950408889634


def _body(dist_ref, pt_ref, types_ref, a_ref, xi_ref, p_ref, q_ref, r0_ref,
          off_ref, ca_ref, cb_ref, out_ref, epa_ref, coef_s):
    b = pl.program_id(0)

    av = a_ref[...]
    pv = p_ref[...]
    qv = q_ref[...]
    r0v = r0_ref[...]
    xiv = xi_ref[...]
    cav = ca_ref[...]
    inv_ba = 1.0 / (cb_ref[...] - cav)
    coef_s[0:1, :] = jnp.log2(0.5 * av) + pv * _ILN2
    coef_s[1:2, :] = (pv / r0v) * _ILN2
    coef_s[2:3, :] = 2.0 * jnp.log2(xiv) + 2.0 * qv * _ILN2
    coef_s[3:4, :] = (2.0 * qv / r0v) * _ILN2
    coef_s[4:5, :] = cav * inv_ba
    coef_s[5:6, :] = inv_ba

    d = dist_ref[0]
    pt = pt_ref[0]
    is1 = pt == 1
    is2 = pt == 2

    def sel(i):
        return jnp.where(is1, coef_s[i, 1],
                         jnp.where(is2, coef_s[i, 2], coef_s[i, 0]))

    x = jnp.clip(sel(5) * d - sel(4), 0.0, 1.0)
    x3 = x * x * x
    fc = 1.0 - x3 * (x * (6.0 * x - 15.0) + 10.0)

    half_phi = jnp.exp2(sel(0) - sel(1) * d) * fc
    rho_e = jnp.exp2(sel(2) - sel(3) * d) * fc

    ones = jnp.ones((1, _M), jnp.float32)
    dims = (((1,), (0,)), ((), ()))
    phi_row = jax.lax.dot_general(
        ones, half_phi, dims, preferred_element_type=jnp.float32)
    half_phi_sum = jnp.sum(phi_row)
    s = jax.lax.dot_general(
        ones, rho_e, dims, preferred_element_type=jnp.float32) + 1e-12

    trow = types_ref[0]
    off_vec = jnp.where(trow == 1, off_ref[0, 1], off_ref[0, 0])
    emb_off_sum = jnp.sum(s * jax.lax.rsqrt(s) - off_vec)

    e = half_phi_sum - emb_off_sum
    bmask = jax.lax.broadcasted_iota(jnp.int32, (1, _B), 1) == b
    erow = jnp.where(bmask, e, out_ref[...])
    out_ref[...] = erow

    @pl.when(b == _B - 1)
    def _fin():
        epa_ref[...] = erow * (1.0 / _N)


def kernel(types, pair_types, distances, A, xi, p, q, r0, offset, cut_a, cut_b):
    dist_t = distances.transpose(0, 2, 1)
    pt_t = pair_types.transpose(0, 2, 1)
    types3 = types.reshape(_B, 1, _N)

    full = lambda shape: pl.BlockSpec(shape, lambda b: tuple(0 for _ in shape))
    energy, energy_per_atom = pl.pallas_call(
        _body,
        grid=(_B,),
        in_specs=[
            pl.BlockSpec((1, _M, _N), lambda b: (b, 0, 0)),
            pl.BlockSpec((1, _M, _N), lambda b: (b, 0, 0)),
            pl.BlockSpec((1, 1, _N), lambda b: (b, 0, 0)),
            full((1, 3)), full((1, 3)), full((1, 3)), full((1, 3)),
            full((1, 3)), full((1, 2)), full((1, 3)), full((1, 3)),
        ],
        out_specs=[
            full((1, _B)),
            full((1, _B)),
        ],
        out_shape=[
            jax.ShapeDtypeStruct((1, _B), jnp.float32),
            jax.ShapeDtypeStruct((1, _B), jnp.float32),
        ],
        scratch_shapes=[pltpu.VMEM((7, 3), jnp.float32)],
    )(dist_t, pt_t, types3,
      A.reshape(1, 3), xi.reshape(1, 3), p.reshape(1, 3), q.reshape(1, 3),
      r0.reshape(1, 3), offset.reshape(1, 2), cut_a.reshape(1, 3),
      cut_b.reshape(1, 3))

    return (energy.T, energy_per_atom.T)
